# Initial kernel scaffold; baseline (speedup 1.0000x reference)
import jax
import jax.numpy as jnp
from jax.experimental import pallas as pl


def kernel(x, edge_index, att1, b1, att2, b2, att3, b3):
    raise NotImplementedError("write your pallas kernel here")


# speedup vs baseline: 38.1367x; 38.1367x over previous
import functools

import jax
import jax.numpy as jnp
from jax import lax
from jax.experimental import pallas as pl
from jax.experimental.pallas import tpu as pltpu
from jax.experimental.pallas import tpu_sc as plsc

N = 10000
D = 128
H = 2
E = 320000
EV = E + N
NEG = 0.2

NC = 2
NS = 16
NW = NC * NS
L = 16
CB = 128
NG = CB // L
NR = D // L
TPW = 10368
NCH = TPW // CB
EPAD = NW * TPW
RPT = 632
NP = NS * RPT

_mesh = plsc.VectorSubcoreMesh(
    core_axis_name="c", subcore_axis_name="s", num_cores=NC, num_subcores=NS
)


_GDN = lax.GatherDimensionNumbers(
    offset_dims=(), collapsed_slice_dims=(0,), start_index_map=(0,))


def _shuf(v, perm):
    return lax.gather(v, perm, _GDN, (1,),
                      mode=lax.GatherScatterMode.PROMISE_IN_BOUNDS)


def _tree_sum16(vecs, iota):
    sh = 1
    while len(vecs) > 1:
        perm = (iota ^ sh).reshape(L, 1)
        sel = (iota & sh) == 0
        nxt = []
        for k in range(0, len(vecs), 2):
            x, y = vecs[k], vecs[k + 1]
            nxt.append(jnp.where(sel, x + _shuf(x, perm), y + _shuf(y, perm)))
        vecs = nxt
        sh *= 2
    return vecs[0]


def _p1_body(h_hbm, srcp, dstp, att_hbm, ex0_hbm, ex1_hbm, dpart_hbm,
             src_v, dst_v, xi, xj, att_v, ex0_v, ex1_v, den_loc,
             sem1, sem2):
    c = lax.axis_index("c")
    s = lax.axis_index("s")
    wid = c * NS + s
    tbase = wid * TPW

    pltpu.sync_copy(att_hbm, att_v)

    z16 = jnp.zeros((L,), jnp.float32)

    @pl.loop(0, 2 * N // L)
    def _zero_den(i):
        den_loc[pl.ds(i * L, L)] = z16

    iota = lax.iota(jnp.int32, L)
    att0r = [att_v[0, pl.ds(r * L, L)] for r in range(NR)]
    att1r = [att_v[1, pl.ds(r * L, L)] for r in range(NR)]

    @pl.loop(0, NCH)
    def _chunk(k):
        base = tbase + k * CB
        pltpu.sync_copy(srcp.at[pl.ds(base, CB)], src_v)
        pltpu.sync_copy(dstp.at[pl.ds(base, CB)], dst_v)
        cpi = pltpu.async_copy(h_hbm.at[dst_v], xi, sem1)
        cpj = pltpu.async_copy(h_hbm.at[src_v], xj, sem2)
        cpi.wait()
        cpj.wait()

        @pl.loop(0, NG)
        def _group(g):
            accs0, accs1 = [], []
            for j in range(L):
                e = g * L + j
                acc0 = acc1 = None
                for r in range(NR):
                    v = xi[e, pl.ds(r * L, L)] + xj[e, pl.ds(r * L, L)]
                    m = jnp.maximum(v, 0.0) + NEG * jnp.minimum(v, 0.0)
                    t0 = m * att0r[r]
                    t1 = m * att1r[r]
                    acc0 = t0 if acc0 is None else acc0 + t0
                    acc1 = t1 if acc1 is None else acc1 + t1
                accs0.append(acc0)
                accs1.append(acc1)
            a0 = _tree_sum16(accs0, iota)
            a1 = _tree_sum16(accs1, iota)
            gid = base + g * L + iota
            mf = jnp.where(gid < EV, 1.0, 0.0).astype(jnp.float32)
            e0 = jnp.exp(a0) * mf
            e1 = jnp.exp(a1) * mf
            ex0_v[pl.ds(g * L, L)] = e0
            ex1_v[pl.ds(g * L, L)] = e1
            dstv = dst_v[pl.ds(g * L, L)]
            plsc.addupdate_scatter(den_loc, [dstv * 2], e0)
            plsc.addupdate_scatter(den_loc, [dstv * 2 + 1], e1)

        pltpu.sync_copy(ex0_v, ex0_hbm.at[pl.ds(base, CB)])
        pltpu.sync_copy(ex1_v, ex1_hbm.at[pl.ds(base, CB)])

    pltpu.sync_copy(den_loc, dpart_hbm.at[pl.ds(wid * 2 * N, 2 * N)])


_pass1 = pl.kernel(
    _p1_body,
    out_type=[
        jax.ShapeDtypeStruct((EPAD,), jnp.float32),
        jax.ShapeDtypeStruct((EPAD,), jnp.float32),
        jax.ShapeDtypeStruct((NW * 2 * N,), jnp.float32),
    ],
    mesh=_mesh,
    compiler_params=pltpu.CompilerParams(needs_layout_passes=False),
    scratch_types=[
        pltpu.VMEM((CB,), jnp.int32),
        pltpu.VMEM((CB,), jnp.int32),
        pltpu.VMEM((CB, D), jnp.float32),
        pltpu.VMEM((CB, D), jnp.float32),
        pltpu.VMEM((H, D), jnp.float32),
        pltpu.VMEM((CB,), jnp.float32),
        pltpu.VMEM((CB,), jnp.float32),
        pltpu.VMEM((2 * N,), jnp.float32),
        pltpu.SemaphoreType.DMA,
        pltpu.SemaphoreType.DMA,
    ],
)


def _p2_body(h_hbm, srcp, dstp, ex0_hbm, ex1_hbm, den_hbm,
             alpha0_hbm, alpha1_hbm, opart_hbm,
             src_v, dst_v, ex0_v, ex1_v, den_v, a0_v, a1_v, msg, osh,
             sem1):
    c = lax.axis_index("c")
    s = lax.axis_index("s")
    wid = c * NS + s
    tbase = wid * TPW

    pltpu.sync_copy(den_hbm, den_v)

    z16 = jnp.zeros((L,), jnp.float32)

    @pl.loop(0, CB)
    def _zero_msg(j):
        for r in range(NR):
            msg[j, pl.ds(r * L, L)] = z16

    for off, sz in ((0, 128), (128, 128), (256, 128), (384, 128), (512, 120)):
        pltpu.sync_copy(msg.at[pl.ds(0, sz)],
                        osh.at[pl.ds(s * RPT + off, sz)])
    plsc.subcore_barrier()

    @pl.loop(0, NCH)
    def _chunk(k):
        base = tbase + k * CB
        pltpu.sync_copy(srcp.at[pl.ds(base, CB)], src_v)
        pltpu.sync_copy(dstp.at[pl.ds(base, CB)], dst_v)
        pltpu.sync_copy(ex0_hbm.at[pl.ds(base, CB)], ex0_v)
        pltpu.sync_copy(ex1_hbm.at[pl.ds(base, CB)], ex1_v)
        cpj = pltpu.async_copy(h_hbm.at[src_v], msg, sem1)
        cpj.wait()

        @pl.loop(0, NG)
        def _group(g):
            dstv = dst_v[pl.ds(g * L, L)]
            e0 = ex0_v[pl.ds(g * L, L)]
            e1 = ex1_v[pl.ds(g * L, L)]
            d0 = plsc.load_gather(den_v, [dstv * 2])
            d1 = plsc.load_gather(den_v, [dstv * 2 + 1])
            a0 = e0 / (d0 + 1e-16)
            a1 = e1 / (d1 + 1e-16)
            a0_v[pl.ds(g * L, L)] = a0
            a1_v[pl.ds(g * L, L)] = a1
            w = (a0 + a1) * 0.5
            for j in range(L):
                e = g * L + j
                ws = w[j]
                for r in range(NR):
                    msg[e, pl.ds(r * L, L)] = msg[e, pl.ds(r * L, L)] * ws

        pltpu.sync_copy(a0_v, alpha0_hbm.at[pl.ds(base, CB)])
        pltpu.sync_copy(a1_v, alpha1_hbm.at[pl.ds(base, CB)])
        pltpu.sync_copy(msg, osh.at[dst_v], add=True)

    plsc.subcore_barrier()
    pltpu.sync_copy(osh.at[pl.ds(s * RPT, RPT)],
                    opart_hbm.at[pl.ds(c * NP + s * RPT, RPT)])


_pass2 = pl.kernel(
    _p2_body,
    out_type=[
        jax.ShapeDtypeStruct((EPAD,), jnp.float32),
        jax.ShapeDtypeStruct((EPAD,), jnp.float32),
        jax.ShapeDtypeStruct((NC * NP, D), jnp.float32),
    ],
    mesh=_mesh,
    compiler_params=pltpu.CompilerParams(needs_layout_passes=False),
    scratch_types=[
        pltpu.VMEM((CB,), jnp.int32),
        pltpu.VMEM((CB,), jnp.int32),
        pltpu.VMEM((CB,), jnp.float32),
        pltpu.VMEM((CB,), jnp.float32),
        pltpu.VMEM((N * 2,), jnp.float32),
        pltpu.VMEM((CB,), jnp.float32),
        pltpu.VMEM((CB,), jnp.float32),
        pltpu.VMEM((CB, D), jnp.float32),
        pltpu.VMEM_SHARED((NP, D), jnp.float32),
        pltpu.SemaphoreType.DMA,
    ],
)


def _dcomb_body(dp_ref, out_ref):
    out_ref[...] = jnp.sum(dp_ref[...], axis=0, keepdims=True)


_dcomb = pl.pallas_call(
    _dcomb_body,
    out_shape=jax.ShapeDtypeStruct((1, 2 * N), jnp.float32),
)


def _hcomb_body(scale, p_ref, b_ref, acc_ref, h_ref, accout_ref):
    hv = p_ref[0:N, :] + p_ref[NP:NP + N, :] + b_ref[...]
    h_ref[...] = hv
    accout_ref[...] = (acc_ref[...] + hv) * scale


def _make_hcomb(scale):
    return pl.pallas_call(
        functools.partial(_hcomb_body, scale),
        out_shape=[
            jax.ShapeDtypeStruct((N, D), jnp.float32),
            jax.ShapeDtypeStruct((N, D), jnp.float32),
        ],
    )


_hcomb_mid = _make_hcomb(1.0)
_hcomb_last = _make_hcomb(0.25)


def kernel(x, edge_index, att1, b1, att2, b2, att3, b3):
    loops = jnp.arange(N, dtype=edge_index.dtype)
    pad = jnp.zeros((EPAD - EV,), edge_index.dtype)
    srcp = jnp.concatenate([edge_index[0], loops, pad])
    dstp = jnp.concatenate([edge_index[1], loops, pad])

    h = x
    acc = x
    alphas = []
    for li, (att, b) in enumerate(((att1, b1), (att2, b2), (att3, b3))):
        att2d = att.reshape(H, D)
        ex0, ex1, dpart = _pass1(h, srcp, dstp, att2d)
        den = _dcomb(dpart.reshape(NW, 2 * N))
        a0p, a1p, opart = _pass2(h, srcp, dstp, ex0, ex1, den.reshape(-1))
        comb = _hcomb_last if li == 2 else _hcomb_mid
        h, acc = comb(opart, b.reshape(1, D), acc)
        alphas.append(jnp.stack([a0p[:EV], a1p[:EV]], axis=1))
    return (acc, alphas[0], alphas[1], alphas[2])
